# Initial kernel scaffold; baseline (speedup 1.0000x reference)
import jax
import jax.numpy as jnp
from jax.experimental import pallas as pl


def kernel(q, k, e, r):
    raise NotImplementedError("write your pallas kernel here")


# speedup vs baseline: 20.5540x; 20.5540x over previous
import functools
import math

import jax
import jax.numpy as jnp
from jax import lax
from jax.experimental import pallas as pl
from jax.experimental.pallas import tpu as pltpu
from jax.experimental.pallas import tpu_sc as plsc

NCORES = 2
NSUB = 16
NW = NCORES * NSUB
LANES = 16
EPS = 1e-16

H = 8
KEY_DIM = 32
HPAD = KEY_DIM + 1
T1 = 80
T2 = 128


def _mesh():
    return plsc.VectorSubcoreMesh(
        core_axis_name="c", subcore_axis_name="s",
        num_cores=NCORES, num_subcores=NSUB)


_CPARAMS = pltpu.CompilerParams(use_tc_tiling_on_sc=False,
                                needs_layout_passes=False)


def _lane_col_base(row_words):
    lane = lax.iota(jnp.int32, LANES)
    return (lane >> 3) * (8 * HPAD) + (lane & 7) * HPAD


@functools.partial(jax.jit, static_argnums=(5, 6))
def _sc_logits(q2, k2, e0, e1, r, n_nodes, n_edges):
    N = n_nodes
    M = n_edges
    RW = 2 * 8 * HPAD
    EPW = M // NW
    NFULL = EPW // T1
    TAIL = EPW - NFULL * T1
    ZRA = (N // NSUB) & ~7
    ZLAST = N - ZRA * (NSUB - 1)
    scale = 1.0 / math.sqrt(float(KEY_DIM))

    def body(q_hbm, k_hbm, e0_hbm, e1_hbm, r_hbm,
             p_hbm, part0_hbm, part1_hbm,
             e0b, e1b, rb, e0t, e1t, rbt, qb, kb, pb, zb, denom, sem):
        cid = lax.axis_index("c")
        sid = lax.axis_index("s")
        w = cid * NSUB + sid
        cbase = _lane_col_base(RW)

        def zrow(i, carry):
            zb[i, :] = jnp.zeros((LANES,), jnp.float32)
            return carry
        lax.fori_loop(0, ZLAST, zrow, 0)

        @pl.when(sid < NSUB - 1)
        def _():
            pltpu.sync_copy(zb.at[pl.ds(0, ZRA)], denom.at[pl.ds(sid * ZRA, ZRA)])

        @pl.when(sid == NSUB - 1)
        def _():
            pltpu.sync_copy(zb, denom.at[pl.ds((NSUB - 1) * ZRA, ZLAST)])

        plsc.subcore_barrier()

        def do_chunk(base, tc, e0r, e1r, rr):
            pltpu.sync_copy(e0_hbm.at[pl.ds(base, tc)], e0r)
            pltpu.sync_copy(e1_hbm.at[pl.ds(base, tc)], e1r)
            pltpu.sync_copy(r_hbm.at[pl.ds(base, tc)], rr)
            dq = pltpu.async_copy(q_hbm.at[e0r], qb.at[pl.ds(0, tc)], sem)
            dk = pltpu.async_copy(k_hbm.at[e1r], kb.at[pl.ds(0, tc)], sem)
            dq.wait()
            dk.wait()

            def edge(m, carry):
                rowv = jnp.full((LANES,), m, jnp.int32)
                acc = jnp.zeros((LANES,), jnp.float32)
                for j in range(KEY_DIM):
                    col = cbase + j
                    qv = plsc.load_gather(qb, [rowv, col])
                    kv = plsc.load_gather(kb, [rowv, col])
                    acc = acc + qv * kv
                pb[m, :] = jnp.exp(acc * scale)
                return carry
            lax.fori_loop(0, tc, edge, 0)

            pltpu.sync_copy(pb.at[pl.ds(0, tc)], p_hbm.at[pl.ds(base, tc)])
            pltpu.sync_copy(pb.at[pl.ds(0, tc)], denom.at[rr], add=True)

        def chunk_loop(c, carry):
            do_chunk(w * EPW + c * T1, T1, e0b, e1b, rb)
            return carry
        lax.fori_loop(0, NFULL, chunk_loop, 0)
        if TAIL:
            do_chunk(w * EPW + NFULL * T1, TAIL, e0t, e1t, rbt)

        plsc.subcore_barrier()

        def copy_out(off, rows):
            pltpu.sync_copy(denom.at[pl.ds(off, rows)], zb.at[pl.ds(0, rows)])

            @pl.when(cid == 0)
            def _():
                pltpu.sync_copy(zb.at[pl.ds(0, rows)],
                                part0_hbm.at[pl.ds(off, rows)])

            @pl.when(cid == 1)
            def _():
                pltpu.sync_copy(zb.at[pl.ds(0, rows)],
                                part1_hbm.at[pl.ds(off, rows)])

        @pl.when(sid < NSUB - 1)
        def _():
            copy_out(sid * ZRA, ZRA)

        @pl.when(sid == NSUB - 1)
        def _():
            copy_out((NSUB - 1) * ZRA, ZLAST)

    f = pl.kernel(
        body,
        out_type=[
            jax.ShapeDtypeStruct((M, LANES), jnp.float32),
            jax.ShapeDtypeStruct((N, LANES), jnp.float32),
            jax.ShapeDtypeStruct((N, LANES), jnp.float32),
        ],
        mesh=_mesh(),
        compiler_params=_CPARAMS,
        scratch_types=[
            pltpu.VMEM((T1,), jnp.int32),
            pltpu.VMEM((T1,), jnp.int32),
            pltpu.VMEM((T1,), jnp.int32),
            pltpu.VMEM((max(TAIL, 8),), jnp.int32),
            pltpu.VMEM((max(TAIL, 8),), jnp.int32),
            pltpu.VMEM((max(TAIL, 8),), jnp.int32),
            pltpu.VMEM((T1, RW), jnp.float32),
            pltpu.VMEM((T1, RW), jnp.float32),
            pltpu.VMEM((T1, LANES), jnp.float32),
            pltpu.VMEM((ZLAST, LANES), jnp.float32),
            pltpu.VMEM_SHARED((N, LANES), jnp.float32),
            pltpu.SemaphoreType.DMA,
        ],
    )
    return f(q2, k2, e0, e1, r)


@functools.partial(jax.jit, static_argnums=(4, 5))
def _sc_normalize(p2d, part0, part1, r, n_nodes, n_edges):
    M = n_edges
    EPW = M // NW
    NFULL = EPW // T2
    TAIL = EPW - NFULL * T2

    def body(p_hbm, d0_hbm, d1_hbm, r_hbm, o_hbm,
             rb, rbt, pb, g0b, g1b, ob, sem):
        cid = lax.axis_index("c")
        sid = lax.axis_index("s")
        w = cid * NSUB + sid

        def do_chunk(base, tc, rr):
            pltpu.sync_copy(r_hbm.at[pl.ds(base, tc)], rr)
            pltpu.sync_copy(p_hbm.at[pl.ds(base, tc)], pb.at[pl.ds(0, tc)])
            d0 = pltpu.async_copy(d0_hbm.at[rr], g0b.at[pl.ds(0, tc)], sem)
            d1 = pltpu.async_copy(d1_hbm.at[rr], g1b.at[pl.ds(0, tc)], sem)
            d0.wait()
            d1.wait()

            def row(t, carry):
                ob[t, :] = pb[t, :] / (g0b[t, :] + g1b[t, :] + EPS)
                return carry
            lax.fori_loop(0, tc, row, 0)
            pltpu.sync_copy(ob.at[pl.ds(0, tc)], o_hbm.at[pl.ds(base, tc)])

        def chunk_loop(c, carry):
            do_chunk(w * EPW + c * T2, T2, rb)
            return carry
        lax.fori_loop(0, NFULL, chunk_loop, 0)
        if TAIL:
            do_chunk(w * EPW + NFULL * T2, TAIL, rbt)

    f = pl.kernel(
        body,
        out_type=jax.ShapeDtypeStruct((M, LANES), jnp.float32),
        mesh=_mesh(),
        compiler_params=_CPARAMS,
        scratch_types=[
            pltpu.VMEM((T2,), jnp.int32),
            pltpu.VMEM((max(TAIL, 8),), jnp.int32),
            pltpu.VMEM((T2, LANES), jnp.float32),
            pltpu.VMEM((T2, LANES), jnp.float32),
            pltpu.VMEM((T2, LANES), jnp.float32),
            pltpu.VMEM((T2, LANES), jnp.float32),
            pltpu.SemaphoreType.DMA,
        ],
    )
    return f(p2d, part0, part1, r)


def kernel(q, k, e, r):
    B, N, D = q.shape
    M = e.shape[1]
    assert B == 2 and D == H * KEY_DIM and M % NW == 0

    def relayout(x):
        xt = jnp.moveaxis(x, 0, 1).reshape(N, B, H, KEY_DIM)
        xp = jnp.pad(xt, ((0, 0), (0, 0), (0, 0), (0, HPAD - KEY_DIM)))
        return xp.reshape(N, B * H * HPAD)

    q2 = relayout(q)
    k2 = relayout(k)
    p2d, part0, part1 = _sc_logits(q2, k2, e[0], e[1], r, N, M)
    o2d = _sc_normalize(p2d, part0, part1, r, N, M)
    return jnp.moveaxis(o2d.reshape(M, B, H), 0, 1)


# speedup vs baseline: 25.9631x; 1.2632x over previous
import functools
import math

import jax
import jax.numpy as jnp
from jax import lax
from jax.experimental import pallas as pl
from jax.experimental.pallas import tpu as pltpu
from jax.experimental.pallas import tpu_sc as plsc

NCORES = 2
NSUB = 16
NW = NCORES * NSUB
LANES = 16
EPS = 1e-16

H = 8
KEY_DIM = 32
T1 = 40
GRP = 16
T2 = 125


def _mesh():
    return plsc.VectorSubcoreMesh(
        core_axis_name="c", subcore_axis_name="s",
        num_cores=NCORES, num_subcores=NSUB)


_CPARAMS = pltpu.CompilerParams(use_tc_tiling_on_sc=False,
                                needs_layout_passes=False)


@functools.partial(jax.jit, static_argnums=(7, 8))
def _sc_logits(q0, q1, k0, k1, e0r, e1r, rr, n_nodes, n_edges):
    N = n_nodes
    M = n_edges
    D = H * KEY_DIM
    EPW = M // NW
    NCH = EPW // T1
    NPAIR = NCH // 2
    NGRP = NCH // GRP
    GTAIL = NCH - NGRP * GRP
    ZRA = (N // NSUB) & ~7
    ZLAST = N - ZRA * (NSUB - 1)
    scale = 1.0 / math.sqrt(float(KEY_DIM))

    def body(q0_hbm, q1_hbm, k0_hbm, k1_hbm, e0r_hbm, e1r_hbm, rr_hbm,
             p_hbm, part0_hbm, part1_hbm,
             e0a, e1a, ra, qbA, kbA, qbB, kbB, pb, denom,
             semA, semB, semf, semp):
        cid = lax.axis_index("c")
        sid = lax.axis_index("s")
        w = cid * NSUB + sid
        lane = lax.iota(jnp.int32, LANES)
        crow = (lane >> 3) * T1
        chh = (lane & 7) * KEY_DIM
        coff0 = lane

        pltpu.sync_copy(e0r_hbm.at[w], e0a)
        pltpu.sync_copy(e1r_hbm.at[w], e1a)
        pltpu.sync_copy(rr_hbm.at[w], ra)

        def zrow(i, carry):
            pb[i, :] = jnp.zeros((LANES,), jnp.float32)
            return carry
        lax.fori_loop(0, ZLAST, zrow, 0)

        @pl.when(sid < NSUB - 1)
        def _():
            pltpu.sync_copy(pb.at[pl.ds(0, ZRA)], denom.at[pl.ds(sid * ZRA, ZRA)])

        @pl.when(sid == NSUB - 1)
        def _():
            pltpu.sync_copy(pb, denom.at[pl.ds((NSUB - 1) * ZRA, ZLAST)])

        plsc.subcore_barrier()

        def g4(c, qbx, kbx, sem):
            i0 = e0a.at[c]
            i1 = e1a.at[c]
            return [
                pltpu.make_async_copy(q0_hbm.at[i0], qbx.at[pl.ds(0, T1)], sem),
                pltpu.make_async_copy(q1_hbm.at[i0], qbx.at[pl.ds(T1, T1)], sem),
                pltpu.make_async_copy(k0_hbm.at[i1], kbx.at[pl.ds(0, T1)], sem),
                pltpu.make_async_copy(k1_hbm.at[i1], kbx.at[pl.ds(T1, T1)], sem),
            ]

        def issue(c, qbx, kbx, sem):
            for d in g4(c, qbx, kbx, sem):
                d.start()

        def wait(c, qbx, kbx, sem):
            for d in g4(c, qbx, kbx, sem):
                d.wait()

        def compute(c, qbx, kbx):
            slot = lax.rem(c, GRP) * T1

            def edge(m, carry):
                rowv = crow + m
                acc = jnp.zeros((LANES,), jnp.float32)
                off = coff0
                for _ in range(KEY_DIM):
                    col = chh + off
                    qv = plsc.load_gather(qbx, [rowv, col])
                    kv = plsc.load_gather(kbx, [rowv, col])
                    acc = acc + qv * kv
                    off = (off + 1) & (KEY_DIM - 1)
                pb[slot + m, :] = jnp.exp(acc * scale)
                return carry
            lax.fori_loop(0, T1, edge, 0)

        def flush(c_last, nch):
            gbase = w * EPW + (c_last - (nch - 1)) * T1
            ds = [pltpu.make_async_copy(
                pb.at[pl.ds(0, nch * T1)],
                p_hbm.at[pl.ds(gbase, nch * T1)], semp)]
            ds[0].start()
            for cc in range(nch):
                d = pltpu.async_copy(
                    pb.at[pl.ds(cc * T1, T1)],
                    denom.at[ra.at[c_last - (nch - 1) + cc]],
                    semf, add=True)
                ds.append(d)
            for d in ds:
                d.wait()

        issue(0, qbA, kbA, semA)

        def pair(i, carry):
            cA = i * 2
            wait(cA, qbA, kbA, semA)
            issue(cA + 1, qbB, kbB, semB)
            compute(cA, qbA, kbA)

            cB = cA + 1
            wait(cB, qbB, kbB, semB)

            @pl.when(cB + 1 < NCH)
            def _():
                issue(cB + 1, qbA, kbA, semA)
            compute(cB, qbB, kbB)

            @pl.when(lax.rem(cB, GRP) == GRP - 1)
            def _():
                flush(cB, GRP)
            return carry
        lax.fori_loop(0, NPAIR, pair, 0)

        cE = NCH - 1
        wait(cE, qbA, kbA, semA)
        compute(cE, qbA, kbA)
        flush(cE, GTAIL)

        plsc.subcore_barrier()

        def copy_out(off, rows):
            pltpu.sync_copy(denom.at[pl.ds(off, rows)], pb.at[pl.ds(0, rows)])

            @pl.when(cid == 0)
            def _():
                pltpu.sync_copy(pb.at[pl.ds(0, rows)],
                                part0_hbm.at[pl.ds(off, rows)])

            @pl.when(cid == 1)
            def _():
                pltpu.sync_copy(pb.at[pl.ds(0, rows)],
                                part1_hbm.at[pl.ds(off, rows)])

        @pl.when(sid < NSUB - 1)
        def _():
            copy_out(sid * ZRA, ZRA)

        @pl.when(sid == NSUB - 1)
        def _():
            copy_out((NSUB - 1) * ZRA, ZLAST)

    f = pl.kernel(
        body,
        out_type=[
            jax.ShapeDtypeStruct((M, LANES), jnp.float32),
            jax.ShapeDtypeStruct((N, LANES), jnp.float32),
            jax.ShapeDtypeStruct((N, LANES), jnp.float32),
        ],
        mesh=_mesh(),
        compiler_params=_CPARAMS,
        scratch_types=[
            pltpu.VMEM((NCH, T1), jnp.int32),
            pltpu.VMEM((NCH, T1), jnp.int32),
            pltpu.VMEM((NCH, T1), jnp.int32),
            pltpu.VMEM((2 * T1, D), jnp.float32),
            pltpu.VMEM((2 * T1, D), jnp.float32),
            pltpu.VMEM((2 * T1, D), jnp.float32),
            pltpu.VMEM((2 * T1, D), jnp.float32),
            pltpu.VMEM((GRP * T1, LANES), jnp.float32),
            pltpu.VMEM_SHARED((N, LANES), jnp.float32),
            pltpu.SemaphoreType.DMA,
            pltpu.SemaphoreType.DMA,
            pltpu.SemaphoreType.DMA,
            pltpu.SemaphoreType.DMA,
        ],
    )
    return f(q0, q1, k0, k1, e0r, e1r, rr)


@functools.partial(jax.jit, static_argnums=(4, 5))
def _sc_normalize(p2d, part0, part1, rr2, n_nodes, n_edges):
    M = n_edges
    EPW = M // NW
    NCH = EPW // T2
    NPAIR = NCH // 2

    def body(p_hbm, d0_hbm, d1_hbm, rr_hbm, o_hbm,
             ra, pbA, g0A, g1A, pbB, g0B, g1B, obA, obB,
             semA, semB, sempA, sempB):
        cid = lax.axis_index("c")
        sid = lax.axis_index("s")
        w = cid * NSUB + sid

        pltpu.sync_copy(rr_hbm.at[w], ra)

        def g3(c, pbx, g0x, g1x, sem, semp):
            base = w * EPW + c * T2
            rc = ra.at[c]
            return [
                pltpu.make_async_copy(
                    p_hbm.at[pl.ds(base * LANES, T2 * LANES)], pbx, semp),
                pltpu.make_async_copy(d0_hbm.at[rc], g0x, sem),
                pltpu.make_async_copy(d1_hbm.at[rc], g1x, sem),
            ]

        def issue(c, pbx, g0x, g1x, sem, semp):
            for d in g3(c, pbx, g0x, g1x, sem, semp):
                d.start()

        def wait(c, pbx, g0x, g1x, sem, semp):
            for d in g3(c, pbx, g0x, g1x, sem, semp):
                d.wait()

        def compute(c, pbx, g0x, g1x, obx):
            def row(t, carry):
                num = pbx[pl.ds(t * LANES, LANES)]
                den = g0x[t, :] + g1x[t, :] + EPS
                obx[pl.ds(t * LANES, LANES)] = num / den
                return carry
            lax.fori_loop(0, T2, row, 0)
            pltpu.sync_copy(
                obx, o_hbm.at[pl.ds((w * EPW + c * T2) * LANES, T2 * LANES)])

        issue(0, pbA, g0A, g1A, semA, sempA)

        def pair(i, carry):
            cA = i * 2
            wait(cA, pbA, g0A, g1A, semA, sempA)
            issue(cA + 1, pbB, g0B, g1B, semB, sempB)
            compute(cA, pbA, g0A, g1A, obA)

            cB = cA + 1
            wait(cB, pbB, g0B, g1B, semB, sempB)

            @pl.when(cB + 1 < NCH)
            def _():
                issue(cB + 1, pbA, g0A, g1A, semA, sempA)
            compute(cB, pbB, g0B, g1B, obB)
            return carry
        lax.fori_loop(0, NPAIR, pair, 0)

    f = pl.kernel(
        body,
        out_type=jax.ShapeDtypeStruct((M * LANES,), jnp.float32),
        mesh=_mesh(),
        compiler_params=_CPARAMS,
        scratch_types=[
            pltpu.VMEM((NCH, T2), jnp.int32),
            pltpu.VMEM((T2 * LANES,), jnp.float32),
            pltpu.VMEM((T2, LANES), jnp.float32),
            pltpu.VMEM((T2, LANES), jnp.float32),
            pltpu.VMEM((T2 * LANES,), jnp.float32),
            pltpu.VMEM((T2, LANES), jnp.float32),
            pltpu.VMEM((T2, LANES), jnp.float32),
            pltpu.VMEM((T2 * LANES,), jnp.float32),
            pltpu.VMEM((T2 * LANES,), jnp.float32),
            pltpu.SemaphoreType.DMA,
            pltpu.SemaphoreType.DMA,
            pltpu.SemaphoreType.DMA,
            pltpu.SemaphoreType.DMA,
        ],
    )
    return f(p2d, part0, part1, rr2)


def kernel(q, k, e, r):
    B, N, D = q.shape
    M = e.shape[1]
    EPW = M // NW
    assert B == 2 and D == H * KEY_DIM
    assert M % NW == 0 and EPW % T1 == 0 and EPW % T2 == 0

    q0, q1 = q[0], q[1]
    k0, k1 = k[0], k[1]
    e0r = e[0].reshape(NW, EPW // T1, T1)
    e1r = e[1].reshape(NW, EPW // T1, T1)
    rr1 = r.reshape(NW, EPW // T1, T1)
    rr2 = r.reshape(NW, EPW // T2, T2)

    p2d, part0, part1 = _sc_logits(q0, q1, k0, k1, e0r, e1r, rr1, N, M)
    of = _sc_normalize(p2d.reshape(M * LANES), part0, part1, rr2, N, M)
    return jnp.moveaxis(of.reshape(M, B, H), 0, 1)


# speedup vs baseline: 37.5829x; 1.4476x over previous
import functools
import math

import jax
import jax.numpy as jnp
from jax import lax
from jax.experimental import pallas as pl
from jax.experimental.pallas import tpu as pltpu
from jax.experimental.pallas import tpu_sc as plsc

NCORES = 2
NSUB = 16
NW = NCORES * NSUB
LANES = 16
EPS = 1e-16

H = 8
KEY_DIM = 32
T1 = 40
GRP = 16
T2 = 40


def _mesh():
    return plsc.VectorSubcoreMesh(
        core_axis_name="c", subcore_axis_name="s",
        num_cores=NCORES, num_subcores=NSUB)


_CPARAMS = pltpu.CompilerParams(use_tc_tiling_on_sc=False,
                                needs_layout_passes=False)


@functools.partial(jax.jit, static_argnums=(4, 5))
def _sc_logits(q, k, er, rr, n_nodes, n_edges):
    N = n_nodes
    M = n_edges
    D = H * KEY_DIM
    EPW = M // NW
    NCH = EPW // T1
    NPAIR = NCH // 2
    NGRP = NCH // GRP
    GTAIL = NCH - NGRP * GRP
    ZRA = (N // NSUB) & ~7
    ZLAST = N - ZRA * (NSUB - 1)
    scale = 1.0 / math.sqrt(float(KEY_DIM))

    def body(q_hbm, k_hbm, er_hbm, rr_hbm,
             p_hbm, part0_hbm, part1_hbm,
             e0a, e1a, ra, qbA, kbA, qbB, kbB, pb, denom,
             semA, semB, semf, semp):
        cid = lax.axis_index("c")
        sid = lax.axis_index("s")
        w = cid * NSUB + sid
        q0_hbm, q1_hbm = q_hbm.at[0], q_hbm.at[1]
        k0_hbm, k1_hbm = k_hbm.at[0], k_hbm.at[1]
        lane = lax.iota(jnp.int32, LANES)
        crow = (lane >> 3) * T1
        chh = (lane & 7) * KEY_DIM
        coff0 = lane

        pltpu.sync_copy(er_hbm.at[0, w], e0a)
        pltpu.sync_copy(er_hbm.at[1, w], e1a)
        pltpu.sync_copy(rr_hbm.at[w], ra)

        def zrow(i, carry):
            pb[i, :] = jnp.zeros((LANES,), jnp.float32)
            return carry
        lax.fori_loop(0, ZLAST, zrow, 0)

        @pl.when(sid < NSUB - 1)
        def _():
            pltpu.sync_copy(pb.at[pl.ds(0, ZRA)], denom.at[pl.ds(sid * ZRA, ZRA)])

        @pl.when(sid == NSUB - 1)
        def _():
            pltpu.sync_copy(pb, denom.at[pl.ds((NSUB - 1) * ZRA, ZLAST)])

        plsc.subcore_barrier()

        def g4(c, qbx, kbx, sem):
            i0 = e0a.at[c]
            i1 = e1a.at[c]
            return [
                pltpu.make_async_copy(q0_hbm.at[i0], qbx.at[pl.ds(0, T1)], sem),
                pltpu.make_async_copy(q1_hbm.at[i0], qbx.at[pl.ds(T1, T1)], sem),
                pltpu.make_async_copy(k0_hbm.at[i1], kbx.at[pl.ds(0, T1)], sem),
                pltpu.make_async_copy(k1_hbm.at[i1], kbx.at[pl.ds(T1, T1)], sem),
            ]

        def issue(c, qbx, kbx, sem):
            for d in g4(c, qbx, kbx, sem):
                d.start()

        def wait(c, qbx, kbx, sem):
            for d in g4(c, qbx, kbx, sem):
                d.wait()

        def compute(c, qbx, kbx):
            slot = lax.rem(c, GRP) * T1

            def edge(m, carry):
                rowv = crow + m
                acc = jnp.zeros((LANES,), jnp.float32)
                off = coff0
                for _ in range(KEY_DIM):
                    col = chh + off
                    qv = plsc.load_gather(qbx, [rowv, col])
                    kv = plsc.load_gather(kbx, [rowv, col])
                    acc = acc + qv * kv
                    off = (off + 1) & (KEY_DIM - 1)
                pb[slot + m, :] = jnp.exp(acc * scale)
                return carry
            lax.fori_loop(0, T1, edge, 0)

        def flush(c_last, nch):
            gbase = w * EPW + (c_last - (nch - 1)) * T1
            ds = [pltpu.make_async_copy(
                pb.at[pl.ds(0, nch * T1)],
                p_hbm.at[pl.ds(gbase, nch * T1)], semp)]
            ds[0].start()
            for cc in range(nch):
                d = pltpu.async_copy(
                    pb.at[pl.ds(cc * T1, T1)],
                    denom.at[ra.at[c_last - (nch - 1) + cc]],
                    semf, add=True)
                ds.append(d)
            for d in ds:
                d.wait()

        issue(0, qbA, kbA, semA)

        def pair(i, carry):
            cA = i * 2
            wait(cA, qbA, kbA, semA)
            issue(cA + 1, qbB, kbB, semB)
            compute(cA, qbA, kbA)

            cB = cA + 1
            wait(cB, qbB, kbB, semB)

            @pl.when(cB + 1 < NCH)
            def _():
                issue(cB + 1, qbA, kbA, semA)
            compute(cB, qbB, kbB)

            @pl.when(lax.rem(cB, GRP) == GRP - 1)
            def _():
                flush(cB, GRP)
            return carry
        lax.fori_loop(0, NPAIR, pair, 0)

        cE = NCH - 1
        wait(cE, qbA, kbA, semA)
        compute(cE, qbA, kbA)
        flush(cE, GTAIL)

        plsc.subcore_barrier()

        def copy_out(off, rows):
            pltpu.sync_copy(denom.at[pl.ds(off, rows)], pb.at[pl.ds(0, rows)])

            @pl.when(cid == 0)
            def _():
                pltpu.sync_copy(pb.at[pl.ds(0, rows)],
                                part0_hbm.at[pl.ds(off, rows)])

            @pl.when(cid == 1)
            def _():
                pltpu.sync_copy(pb.at[pl.ds(0, rows)],
                                part1_hbm.at[pl.ds(off, rows)])

        @pl.when(sid < NSUB - 1)
        def _():
            copy_out(sid * ZRA, ZRA)

        @pl.when(sid == NSUB - 1)
        def _():
            copy_out((NSUB - 1) * ZRA, ZLAST)

    f = pl.kernel(
        body,
        out_type=[
            jax.ShapeDtypeStruct((M, LANES), jnp.float32),
            jax.ShapeDtypeStruct((N, LANES), jnp.float32),
            jax.ShapeDtypeStruct((N, LANES), jnp.float32),
        ],
        mesh=_mesh(),
        compiler_params=_CPARAMS,
        scratch_types=[
            pltpu.VMEM((NCH, T1), jnp.int32),
            pltpu.VMEM((NCH, T1), jnp.int32),
            pltpu.VMEM((NCH, T1), jnp.int32),
            pltpu.VMEM((2 * T1, D), jnp.float32),
            pltpu.VMEM((2 * T1, D), jnp.float32),
            pltpu.VMEM((2 * T1, D), jnp.float32),
            pltpu.VMEM((2 * T1, D), jnp.float32),
            pltpu.VMEM((GRP * T1, LANES), jnp.float32),
            pltpu.VMEM_SHARED((N, LANES), jnp.float32),
            pltpu.SemaphoreType.DMA,
            pltpu.SemaphoreType.DMA,
            pltpu.SemaphoreType.DMA,
            pltpu.SemaphoreType.DMA,
        ],
    )
    return f(q, k, er, rr)


@functools.partial(jax.jit, static_argnums=(4, 5))
def _sc_normalize(p2d, part0, part1, rr, n_nodes, n_edges):
    M = n_edges
    EPW = M // NW
    NCH = EPW // T2
    NPAIR = NCH // 2
    NGRP = NCH // GRP
    GTAIL = NCH - NGRP * GRP

    def body(p_hbm, d0_hbm, d1_hbm, rr_hbm, o_hbm,
             ra, pbA, g0A, g1A, pbB, g0B, g1B, obx,
             semA, semB, sempA, sempB, semw):
        cid = lax.axis_index("c")
        sid = lax.axis_index("s")
        w = cid * NSUB + sid
        lane = lax.iota(jnp.int32, LANES)
        cb = lane >> 3
        cpos = lane & 7

        pltpu.sync_copy(rr_hbm.at[w], ra)

        def g3(c, pbx, g0x, g1x, sem, semp):
            base = w * EPW + c * T2
            rc = ra.at[c]
            return [
                pltpu.make_async_copy(p_hbm.at[pl.ds(base, T2)], pbx, semp),
                pltpu.make_async_copy(d0_hbm.at[rc], g0x, sem),
                pltpu.make_async_copy(d1_hbm.at[rc], g1x, sem),
            ]

        def issue(c, pbx, g0x, g1x, sem, semp):
            for d in g3(c, pbx, g0x, g1x, sem, semp):
                d.start()

        def wait(c, pbx, g0x, g1x, sem, semp):
            for d in g3(c, pbx, g0x, g1x, sem, semp):
                d.wait()

        def compute(c, pbx, g0x, g1x):
            slot = lax.rem(c, GRP) * T2

            def row(t, carry):
                v = pbx[t, :] / (g0x[t, :] + g1x[t, :] + EPS)
                plsc.store_scatter(obx, [cb, cpos + (slot + t) * H], v)
                return carry
            lax.fori_loop(0, T2, row, 0)

        def flush(c_last, nch):
            gbase = (w * EPW + (c_last - (nch - 1)) * T2) * H
            ds = [
                pltpu.make_async_copy(
                    obx.at[0].at[pl.ds(0, nch * T2 * H)],
                    o_hbm.at[0].at[pl.ds(gbase, nch * T2 * H)], semw),
                pltpu.make_async_copy(
                    obx.at[1].at[pl.ds(0, nch * T2 * H)],
                    o_hbm.at[1].at[pl.ds(gbase, nch * T2 * H)], semw),
            ]
            for d in ds:
                d.start()
            for d in ds:
                d.wait()

        issue(0, pbA, g0A, g1A, semA, sempA)

        def pair(i, carry):
            cA = i * 2
            wait(cA, pbA, g0A, g1A, semA, sempA)
            issue(cA + 1, pbB, g0B, g1B, semB, sempB)
            compute(cA, pbA, g0A, g1A)

            cB = cA + 1
            wait(cB, pbB, g0B, g1B, semB, sempB)

            @pl.when(cB + 1 < NCH)
            def _():
                issue(cB + 1, pbA, g0A, g1A, semA, sempA)
            compute(cB, pbB, g0B, g1B)

            @pl.when(lax.rem(cB, GRP) == GRP - 1)
            def _():
                flush(cB, GRP)
            return carry
        lax.fori_loop(0, NPAIR, pair, 0)

        cE = NCH - 1
        wait(cE, pbA, g0A, g1A, semA, sempA)
        compute(cE, pbA, g0A, g1A)
        flush(cE, GTAIL)

    f = pl.kernel(
        body,
        out_type=jax.ShapeDtypeStruct((2, M * H), jnp.float32),
        mesh=_mesh(),
        compiler_params=_CPARAMS,
        scratch_types=[
            pltpu.VMEM((NCH, T2), jnp.int32),
            pltpu.VMEM((T2, LANES), jnp.float32),
            pltpu.VMEM((T2, LANES), jnp.float32),
            pltpu.VMEM((T2, LANES), jnp.float32),
            pltpu.VMEM((T2, LANES), jnp.float32),
            pltpu.VMEM((T2, LANES), jnp.float32),
            pltpu.VMEM((T2, LANES), jnp.float32),
            pltpu.VMEM((2, GRP * T2 * H), jnp.float32),
            pltpu.SemaphoreType.DMA,
            pltpu.SemaphoreType.DMA,
            pltpu.SemaphoreType.DMA,
            pltpu.SemaphoreType.DMA,
            pltpu.SemaphoreType.DMA,
        ],
    )
    return f(p2d, part0, part1, rr)


def kernel(q, k, e, r):
    B, N, D = q.shape
    M = e.shape[1]
    EPW = M // NW
    assert B == 2 and D == H * KEY_DIM
    assert M % NW == 0 and EPW % T1 == 0 and EPW % T2 == 0

    er = e.reshape(2, NW, EPW // T1, T1)
    rr = r.reshape(NW, EPW // T1, T1)

    p2d, part0, part1 = _sc_logits(q, k, er, rr, N, M)
    of = _sc_normalize(p2d, part0, part1, rr, N, M)
    return of.reshape(B, M, H)
